# baseline (speedup 1.0000x reference)
import functools

import jax
import jax.numpy as jnp
from jax import lax
from jax.experimental import pallas as pl
from jax.experimental.pallas import tpu as pltpu
from jax.experimental.pallas import tpu_sc as plsc

NC = 2
NS = 16
NW = NC * NS


def _gather_body(table_hbm, idx_hbm, out_hbm, idx_v, rows_v, sem,
                 *, b_per_w, chunk):
    wid = lax.axis_index("s") * NC + lax.axis_index("c")
    base = wid * b_per_w
    n_chunks = b_per_w // chunk
    for c in range(n_chunks):
        off = base + c * chunk
        pltpu.sync_copy(idx_hbm.at[pl.ds(off, chunk)], idx_v)
        pltpu.async_copy(table_hbm.at[idx_v], rows_v, sem).wait()
        pltpu.sync_copy(rows_v, out_hbm.at[pl.ds(off, chunk)])


def kernel(x, weight):
    B = x.size
    D = weight.shape[1]
    idx = x.reshape(B).astype(jnp.int32)
    b_per_w = B // NW
    chunk = 832

    mesh = plsc.VectorSubcoreMesh(
        core_axis_name="c", subcore_axis_name="s",
        num_cores=NC, num_subcores=NS)

    body = functools.partial(_gather_body, b_per_w=b_per_w, chunk=chunk)
    out = pl.kernel(
        body,
        out_type=jax.ShapeDtypeStruct((B, D), jnp.float32),
        mesh=mesh,
        scratch_types=[
            pltpu.VMEM((chunk,), jnp.int32),
            pltpu.VMEM((chunk, D), jnp.float32),
            pltpu.SemaphoreType.DMA,
        ],
        compiler_params=pltpu.CompilerParams(use_tc_tiling_on_sc=False),
    )(weight, idx)
    return out.reshape(x.shape + (D,))


# speedup vs baseline: 1.0063x; 1.0063x over previous
import functools

import jax
import jax.numpy as jnp
from jax import lax
from jax.experimental import pallas as pl
from jax.experimental.pallas import tpu as pltpu
from jax.experimental.pallas import tpu_sc as plsc

NC = 2
NS = 16
NW = NC * NS


def _gather_body(table_hbm, idx_hbm, out_hbm, idx_v, *bufs_and_sems,
                 b_per_w, chunk, nbuf):
    rows = bufs_and_sems[:nbuf]
    gsem = bufs_and_sems[nbuf:2 * nbuf]
    osem = bufs_and_sems[2 * nbuf:3 * nbuf]
    n_chunks = b_per_w // chunk

    wid = lax.axis_index("s") * NC + lax.axis_index("c")
    base = wid * b_per_w
    pltpu.sync_copy(idx_hbm.at[pl.ds(base, b_per_w)], idx_v)

    def fire_gather(c):
        b = c % nbuf
        return pltpu.async_copy(
            table_hbm.at[idx_v.at[pl.ds(c * chunk, chunk)]], rows[b], gsem[b])

    gd = [fire_gather(b) for b in range(nbuf)]
    od = [None] * nbuf
    for c in range(n_chunks):
        b = c % nbuf
        gd[b].wait()
        od[b] = pltpu.async_copy(
            rows[b], out_hbm.at[pl.ds(base + c * chunk, chunk)], osem[b])
        nxt = c + nbuf
        if nxt < n_chunks:
            od[b].wait()
            gd[b] = fire_gather(nxt)
    for c in range(max(0, n_chunks - nbuf), n_chunks):
        od[c % nbuf].wait()


def kernel(x, weight):
    B = x.size
    D = weight.shape[1]
    idx = x.reshape(B).astype(jnp.int32)
    b_per_w = B // NW
    chunk = 416
    nbuf = 4

    mesh = plsc.VectorSubcoreMesh(
        core_axis_name="c", subcore_axis_name="s",
        num_cores=NC, num_subcores=NS)

    body = functools.partial(_gather_body, b_per_w=b_per_w, chunk=chunk,
                             nbuf=nbuf)
    out = pl.kernel(
        body,
        out_type=jax.ShapeDtypeStruct((B, D), jnp.float32),
        mesh=mesh,
        scratch_types=(
            [pltpu.VMEM((b_per_w,), jnp.int32)]
            + [pltpu.VMEM((chunk, D), jnp.float32) for _ in range(nbuf)]
            + [pltpu.SemaphoreType.DMA for _ in range(2 * nbuf)]
        ),
        compiler_params=pltpu.CompilerParams(use_tc_tiling_on_sc=False),
    )(weight, idx)
    return out.reshape(x.shape + (D,))
